# baseline (speedup 1.0000x reference)
import functools

import jax
import jax.numpy as jnp
from jax import lax
from jax.experimental import pallas as pl
from jax.experimental.pallas import tpu as pltpu
from jax.experimental.pallas import tpu_sc as plsc

N = 100000
NPAD = 102400
E = 1600000
FLAT = 2 * NPAD
SLICE = FLAT // 16
IDXBUF = 2048
Q = 49
TOT = 32 * Q * IDXBUF
IL_ROWS = FLAT // 128
IL_BLK = 80
IL_GRID = IL_ROWS // IL_BLK

_mesh = plsc.VectorSubcoreMesh(
    core_axis_name="c", subcore_axis_name="s", num_cores=2, num_subcores=16
)


@functools.partial(
    pl.kernel,
    out_type=jax.ShapeDtypeStruct((2, FLAT), jnp.float32),
    mesh=_mesh,
    compiler_params=pltpu.CompilerParams(skip_device_barrier=True),
    scratch_types=[
        pltpu.VMEM_SHARED((FLAT,), jnp.float32),
        pltpu.VMEM_SHARED((FLAT,), jnp.float32),
        pltpu.VMEM((IDXBUF,), jnp.int32),
        pltpu.VMEM((IDXBUF,), jnp.int32),
        pltpu.VMEM((IDXBUF,), jnp.float32),
        pltpu.SemaphoreType.DMA,
    ],
)
def _sc_agg(xs_hbm, src_hbm, dst_hbm, zeros_hbm, out_hbm,
            xs_sp, acc_sp, src_v, dst_v, vals_v, sem):
    cid = lax.axis_index("c")
    sid = lax.axis_index("s")
    r0 = sid * SLICE
    w = cid * 16 + sid
    pltpu.sync_copy(xs_hbm.at[pl.ds(r0, SLICE)], xs_sp.at[pl.ds(r0, SLICE)])
    pltpu.sync_copy(zeros_hbm.at[pl.ds(r0, SLICE)], acc_sp.at[pl.ds(r0, SLICE)])
    plsc.subcore_barrier()

    def body(q, carry):
        off = (w * Q + q) * IDXBUF
        pltpu.sync_copy(src_hbm.at[pl.ds(off, IDXBUF)], src_v)
        pltpu.sync_copy(dst_hbm.at[pl.ds(off, IDXBUF)], dst_v)
        pltpu.async_copy(xs_sp.at[src_v], vals_v, sem).wait()
        pltpu.sync_copy(vals_v, acc_sp.at[dst_v], add=True)
        return carry

    lax.fori_loop(0, Q, body, 0)
    plsc.subcore_barrier()
    pltpu.sync_copy(acc_sp.at[pl.ds(r0, SLICE)], out_hbm.at[cid, pl.ds(r0, SLICE)])


def _parity_masks(shape):
    lane = lax.broadcasted_iota(jnp.int32, shape, len(shape) - 1)
    mo = (lane % 2).astype(jnp.float32)
    return 1.0 - mo, mo


def _tc_a_body(degp_ref, x_ref, dinv_ref, xs1_ref):
    deg = degp_ref[0] + degp_ref[1] + 1.0
    dinv = lax.rsqrt(deg)
    dinv_ref[...] = dinv
    xs1_ref[...] = x_ref[...] * dinv


def _tc_b_body(w1_ref, b1_ref, w2_ref, aggp_ref, xs1_ref, dinv_ref, xs2_ref):
    dinv = dinv_ref[...]
    t = dinv * (aggp_ref[0] + aggp_ref[1] + xs1_ref[...])
    me, mo = _parity_masks(t.shape)
    tr = pltpu.roll(t, 127, axis=1)
    tl = pltpu.roll(t, 1, axis=1)
    t0 = me * t + mo * tl
    t1 = me * tr + mo * t
    y0 = jnp.zeros_like(t)
    y1 = jnp.zeros_like(t)
    for j in range(64):
        h = jnp.maximum(t0 * w1_ref[0, j] + t1 * w1_ref[1, j] + b1_ref[j], 0.0)
        y0 = y0 + h * w2_ref[j, 0]
        y1 = y1 + h * w2_ref[j, 1]
    xs2_ref[...] = dinv * (me * y0 + mo * y1)


def _tc_c_body(b2_ref, aggp_ref, xs2_ref, dinv_ref, out_ref):
    me, mo = _parity_masks(xs2_ref.shape)
    out_ref[...] = (dinv_ref[...] * (aggp_ref[0] + aggp_ref[1] + xs2_ref[...])
                    + me * b2_ref[0] + mo * b2_ref[1])


def _il_spec(lead):
    if lead:
        return pl.BlockSpec((*lead, IL_BLK, 128),
                            lambda i: (*([0] * len(lead)), i, 0))
    return pl.BlockSpec((IL_BLK, 128), lambda i: (i, 0))


_SMEM = pl.BlockSpec(memory_space=pltpu.SMEM)
_IL1 = jax.ShapeDtypeStruct((IL_ROWS, 128), jnp.float32)

_tc_a = pl.pallas_call(
    _tc_a_body,
    grid=(IL_GRID,),
    in_specs=[_il_spec((2,)), _il_spec(())],
    out_specs=[_il_spec(()), _il_spec(())],
    out_shape=[_IL1, _IL1],
)

_tc_b = pl.pallas_call(
    _tc_b_body,
    grid=(IL_GRID,),
    in_specs=[_SMEM, _SMEM, _SMEM, _il_spec((2,)), _il_spec(()), _il_spec(())],
    out_specs=[_il_spec(())],
    out_shape=[_IL1],
)

_tc_c = pl.pallas_call(
    _tc_c_body,
    grid=(IL_GRID,),
    in_specs=[_SMEM, _il_spec((2,)), _il_spec(()), _il_spec(())],
    out_specs=[_il_spec(())],
    out_shape=[_IL1],
)


def kernel(x, edge_index, W1, b1, W2, b2):
    src = edge_index[0].astype(jnp.int32)
    dst = edge_index[1].astype(jnp.int32)
    npad_elems = TOT - 2 * E
    pad = 2 * N + (jnp.arange(npad_elems, dtype=jnp.int32) % 4096)
    src2 = jnp.concatenate(
        [jnp.stack([2 * src, 2 * src + 1], axis=1).reshape(-1), pad])
    dst2 = jnp.concatenate(
        [jnp.stack([2 * dst, 2 * dst + 1], axis=1).reshape(-1), pad])
    zeros_flat = jnp.zeros((FLAT,), jnp.float32)
    ones_flat = jnp.ones((FLAT,), jnp.float32)

    degp = _sc_agg(ones_flat, src2, dst2, zeros_flat)
    degp_il = degp.reshape(2, IL_ROWS, 128)
    x_il = jnp.pad(x, ((0, NPAD - N), (0, 0))).reshape(IL_ROWS, 128)
    dinv_il, xs1_il = _tc_a(degp_il, x_il)

    agg1p = _sc_agg(xs1_il.reshape(FLAT), src2, dst2, zeros_flat)
    (xs2_il,) = _tc_b(W1, b1, W2, agg1p.reshape(2, IL_ROWS, 128),
                      xs1_il, dinv_il)

    agg2p = _sc_agg(xs2_il.reshape(FLAT), src2, dst2, zeros_flat)
    (out_il,) = _tc_c(b2, agg2p.reshape(2, IL_ROWS, 128), xs2_il, dinv_il)

    return out_il.reshape(NPAD, 2)[:N]


# speedup vs baseline: 3.5319x; 3.5319x over previous
import functools

import jax
import jax.numpy as jnp
from jax import lax
from jax.experimental import pallas as pl
from jax.experimental.pallas import tpu as pltpu
from jax.experimental.pallas import tpu_sc as plsc

N = 100000
NPAD = 102400
E = 1600000
FLAT = 2 * NPAD
SLICE = FLAT // 16
EROWS = E // 128
EROWS_P = 12544
WROWS = EROWS_P // 32
QB = 49
EB = 1024
IL_ROWS = FLAT // 128
IL_BLK = 80
IL_GRID = IL_ROWS // IL_BLK

_mesh = plsc.VectorSubcoreMesh(
    core_axis_name="c", subcore_axis_name="s", num_cores=2, num_subcores=16
)


@functools.partial(
    pl.kernel,
    out_type=jax.ShapeDtypeStruct((2, FLAT), jnp.float32),
    mesh=_mesh,
    scratch_types=[
        pltpu.VMEM_SHARED((FLAT,), jnp.float32),
        pltpu.VMEM_SHARED((FLAT,), jnp.float32),
        pltpu.VMEM((8, 128), jnp.int32),
        pltpu.VMEM((8, 128), jnp.int32),
        pltpu.VMEM((EB,), jnp.int32),
        pltpu.VMEM((EB,), jnp.int32),
        pltpu.VMEM((EB,), jnp.int32),
        pltpu.VMEM((EB,), jnp.int32),
        pltpu.VMEM((EB,), jnp.float32),
        pltpu.VMEM((EB,), jnp.float32),
        pltpu.SemaphoreType.DMA,
        pltpu.SemaphoreType.DMA,
    ],
)
def _sc_agg(xs_hbm, src_hbm, dst_hbm, zeros_hbm, out_hbm,
            xs_sp, acc_sp, sbuf, dbuf, ie0, ie1, id0, id1, ve, vo,
            sem_e, sem_o):
    cid = lax.axis_index("c")
    sid = lax.axis_index("s")
    r0 = sid * SLICE
    w = cid * 16 + sid
    pltpu.sync_copy(xs_hbm.at[pl.ds(r0, SLICE)], xs_sp.at[pl.ds(r0, SLICE)])
    pltpu.sync_copy(zeros_hbm.at[pl.ds(r0, SLICE)], acc_sp.at[pl.ds(r0, SLICE)])
    plsc.subcore_barrier()

    row0 = w * WROWS

    def body(q, carry):
        r = row0 + q * 8
        pltpu.sync_copy(src_hbm.at[pl.ds(r, 8)], sbuf)
        pltpu.sync_copy(dst_hbm.at[pl.ds(r, 8)], dbuf)
        for j in range(64):
            ri, c = j // 8, (j % 8) * 16
            o = j * 16
            s2 = sbuf[ri, pl.ds(c, 16)]
            s2 = s2 + s2
            ie0[pl.ds(o, 16)] = s2
            ie1[pl.ds(o, 16)] = s2 + 1
            d2 = dbuf[ri, pl.ds(c, 16)]
            d2 = d2 + d2
            id0[pl.ds(o, 16)] = d2
            id1[pl.ds(o, 16)] = d2 + 1
        cpe = pltpu.async_copy(xs_sp.at[ie0], ve, sem_e)
        cpo = pltpu.async_copy(xs_sp.at[ie1], vo, sem_o)
        cpe.wait()
        cpo.wait()
        pltpu.sync_copy(ve, acc_sp.at[id0], add=True)
        pltpu.sync_copy(vo, acc_sp.at[id1], add=True)
        return carry

    lax.fori_loop(0, QB, body, 0)
    plsc.subcore_barrier()
    pltpu.sync_copy(acc_sp.at[pl.ds(r0, SLICE)], out_hbm.at[cid, pl.ds(r0, SLICE)])


def _tc_idx_body(ei_ref, s_ref, d_ref):
    r = lax.broadcasted_iota(jnp.int32, s_ref.shape, 0)
    l = lax.broadcasted_iota(jnp.int32, s_ref.shape, 1)
    dm = N + ((r * 128 + l) % 2048)
    e0 = ei_ref[0]
    e1 = ei_ref[1]
    s_ref[...] = jnp.where(e0 < N, e0, dm)
    d_ref[...] = jnp.where(e1 < N, e1, dm)


def _parity_masks(shape):
    lane = lax.broadcasted_iota(jnp.int32, shape, len(shape) - 1)
    mo = (lane % 2).astype(jnp.float32)
    return 1.0 - mo, mo


def _tc_a_body(degp_ref, x_ref, dinv_ref, xs1_ref):
    deg = degp_ref[0] + degp_ref[1] + 1.0
    dinv = lax.rsqrt(deg)
    dinv_ref[...] = dinv
    xs1_ref[...] = x_ref[...] * dinv


def _tc_b_body(w1_ref, b1_ref, w2_ref, aggp_ref, xs1_ref, dinv_ref, xs2_ref):
    dinv = dinv_ref[...]
    t = dinv * (aggp_ref[0] + aggp_ref[1] + xs1_ref[...])
    me, mo = _parity_masks(t.shape)
    tr = pltpu.roll(t, 127, axis=1)
    tl = pltpu.roll(t, 1, axis=1)
    t0 = me * t + mo * tl
    t1 = me * tr + mo * t
    y0 = jnp.zeros_like(t)
    y1 = jnp.zeros_like(t)
    for j in range(64):
        h = jnp.maximum(t0 * w1_ref[0, j] + t1 * w1_ref[1, j] + b1_ref[j], 0.0)
        y0 = y0 + h * w2_ref[j, 0]
        y1 = y1 + h * w2_ref[j, 1]
    xs2_ref[...] = dinv * (me * y0 + mo * y1)


def _tc_c_body(b2_ref, aggp_ref, xs2_ref, dinv_ref, out_ref):
    me, mo = _parity_masks(xs2_ref.shape)
    out_ref[...] = (dinv_ref[...] * (aggp_ref[0] + aggp_ref[1] + xs2_ref[...])
                    + me * b2_ref[0] + mo * b2_ref[1])


def _il_spec(lead):
    if lead:
        return pl.BlockSpec((*lead, IL_BLK, 128),
                            lambda i: (*([0] * len(lead)), i, 0))
    return pl.BlockSpec((IL_BLK, 128), lambda i: (i, 0))


_SMEM = pl.BlockSpec(memory_space=pltpu.SMEM)
_IL1 = jax.ShapeDtypeStruct((IL_ROWS, 128), jnp.float32)

IDX_BLK = 448
IDX_GRID = EROWS_P // IDX_BLK

_tc_idx = pl.pallas_call(
    _tc_idx_body,
    grid=(IDX_GRID,),
    in_specs=[pl.BlockSpec((2, IDX_BLK, 128), lambda i: (0, i, 0))],
    out_specs=[pl.BlockSpec((IDX_BLK, 128), lambda i: (i, 0))] * 2,
    out_shape=[jax.ShapeDtypeStruct((EROWS_P, 128), jnp.int32)] * 2,
)

_tc_a = pl.pallas_call(
    _tc_a_body,
    grid=(IL_GRID,),
    in_specs=[_il_spec((2,)), _il_spec(())],
    out_specs=[_il_spec(()), _il_spec(())],
    out_shape=[_IL1, _IL1],
)

_tc_b = pl.pallas_call(
    _tc_b_body,
    grid=(IL_GRID,),
    in_specs=[_SMEM, _SMEM, _SMEM, _il_spec((2,)), _il_spec(()), _il_spec(())],
    out_specs=[_il_spec(())],
    out_shape=[_IL1],
)

_tc_c = pl.pallas_call(
    _tc_c_body,
    grid=(IL_GRID,),
    in_specs=[_SMEM, _il_spec((2,)), _il_spec(()), _il_spec(())],
    out_specs=[_il_spec(())],
    out_shape=[_IL1],
)


def kernel(x, edge_index, W1, b1, W2, b2):
    ei3 = edge_index.astype(jnp.int32).reshape(2, EROWS, 128)
    eip = jnp.pad(ei3, ((0, 0), (0, EROWS_P - EROWS), (0, 0)),
                  constant_values=N)
    srcT, dstT = _tc_idx(eip)
    zeros_flat = jnp.zeros((FLAT,), jnp.float32)
    ones_flat = jnp.ones((FLAT,), jnp.float32)

    degp = _sc_agg(ones_flat, srcT, dstT, zeros_flat)
    degp_il = degp.reshape(2, IL_ROWS, 128)
    x_il = jnp.pad(x, ((0, NPAD - N), (0, 0))).reshape(IL_ROWS, 128)
    dinv_il, xs1_il = _tc_a(degp_il, x_il)

    agg1p = _sc_agg(xs1_il.reshape(FLAT), srcT, dstT, zeros_flat)
    (xs2_il,) = _tc_b(W1, b1, W2, agg1p.reshape(2, IL_ROWS, 128),
                      xs1_il, dinv_il)

    agg2p = _sc_agg(xs2_il.reshape(FLAT), srcT, dstT, zeros_flat)
    (out_il,) = _tc_c(b2, agg2p.reshape(2, IL_ROWS, 128), xs2_il, dinv_il)

    return out_il.reshape(NPAD, 2)[:N]


# speedup vs baseline: 3.9433x; 1.1165x over previous
import functools

import jax
import jax.numpy as jnp
from jax import lax
from jax.experimental import pallas as pl
from jax.experimental.pallas import tpu as pltpu
from jax.experimental.pallas import tpu_sc as plsc

N = 100000
NPAD = 102400
E = 1600000
FLAT = 2 * NPAD
SLICE = FLAT // 16
EROWS = E // 128
EROWS_P = 12544
WROWS = EROWS_P // 32
QB = 49
EB = 1024
IL_ROWS = FLAT // 128
IL_BLK = 80
IL_GRID = IL_ROWS // IL_BLK

_mesh = plsc.VectorSubcoreMesh(
    core_axis_name="c", subcore_axis_name="s", num_cores=2, num_subcores=16
)


@functools.partial(
    pl.kernel,
    out_type=jax.ShapeDtypeStruct((2, FLAT), jnp.float32),
    mesh=_mesh,
    scratch_types=[
        pltpu.VMEM_SHARED((FLAT,), jnp.float32),
        pltpu.VMEM_SHARED((FLAT,), jnp.float32),
        pltpu.VMEM((8, 128), jnp.int32),
        pltpu.VMEM((8, 128), jnp.int32),
        pltpu.VMEM((EB,), jnp.int32),
        pltpu.VMEM((EB,), jnp.int32),
        pltpu.VMEM((EB,), jnp.int32),
        pltpu.VMEM((EB,), jnp.int32),
        pltpu.VMEM((EB,), jnp.float32),
        pltpu.VMEM((EB,), jnp.float32),
        pltpu.SemaphoreType.DMA,
        pltpu.SemaphoreType.DMA,
    ],
)
def _sc_agg(xs_hbm, src_hbm, dst_hbm, zeros_hbm, out_hbm,
            xs_sp, acc_sp, sbuf, dbuf, ie0, ie1, id0, id1, ve, vo,
            sem_e, sem_o):
    cid = lax.axis_index("c")
    sid = lax.axis_index("s")
    r0 = sid * SLICE
    w = cid * 16 + sid
    pltpu.sync_copy(xs_hbm.at[pl.ds(r0, SLICE)], xs_sp.at[pl.ds(r0, SLICE)])
    pltpu.sync_copy(zeros_hbm.at[pl.ds(r0, SLICE)], acc_sp.at[pl.ds(r0, SLICE)])
    plsc.subcore_barrier()

    row0 = w * WROWS

    def body(q, carry):
        r = row0 + q * 8
        pltpu.sync_copy(src_hbm.at[pl.ds(r, 8)], sbuf)
        pltpu.sync_copy(dst_hbm.at[pl.ds(r, 8)], dbuf)
        for j in range(64):
            ri, c = j // 8, (j % 8) * 16
            o = j * 16
            s2 = sbuf[ri, pl.ds(c, 16)]
            s2 = s2 + s2
            ie0[pl.ds(o, 16)] = s2
            ie1[pl.ds(o, 16)] = s2 + 1
            d2 = dbuf[ri, pl.ds(c, 16)]
            d2 = d2 + d2
            id0[pl.ds(o, 16)] = d2
            id1[pl.ds(o, 16)] = d2 + 1
        cpe = pltpu.async_copy(xs_sp.at[ie0], ve, sem_e)
        cpo = pltpu.async_copy(xs_sp.at[ie1], vo, sem_o)
        cpe.wait()
        cpo.wait()
        pltpu.sync_copy(ve, acc_sp.at[id0], add=True)
        pltpu.sync_copy(vo, acc_sp.at[id1], add=True)
        return carry

    lax.fori_loop(0, QB, body, 0)
    plsc.subcore_barrier()
    pltpu.sync_copy(acc_sp.at[pl.ds(r0, SLICE)], out_hbm.at[cid, pl.ds(r0, SLICE)])


@functools.partial(
    pl.kernel,
    out_type=jax.ShapeDtypeStruct((2, FLAT), jnp.float32),
    mesh=_mesh,
    scratch_types=[
        pltpu.VMEM_SHARED((FLAT,), jnp.float32),
        pltpu.VMEM((8, 128), jnp.int32),
        pltpu.VMEM((EB,), jnp.int32),
        pltpu.VMEM((EB,), jnp.int32),
        pltpu.VMEM((EB,), jnp.float32),
    ],
)
def _sc_deg(ones_hbm, dst_hbm, zeros_hbm, out_hbm,
            acc_sp, dbuf, id0, id1, ones_v):
    cid = lax.axis_index("c")
    sid = lax.axis_index("s")
    r0 = sid * SLICE
    w = cid * 16 + sid
    pltpu.sync_copy(zeros_hbm.at[pl.ds(r0, SLICE)], acc_sp.at[pl.ds(r0, SLICE)])
    pltpu.sync_copy(ones_hbm.at[pl.ds(0, EB)], ones_v)
    plsc.subcore_barrier()

    row0 = w * WROWS

    def body(q, carry):
        r = row0 + q * 8
        pltpu.sync_copy(dst_hbm.at[pl.ds(r, 8)], dbuf)
        for j in range(64):
            ri, c = j // 8, (j % 8) * 16
            o = j * 16
            d2 = dbuf[ri, pl.ds(c, 16)]
            d2 = d2 + d2
            id0[pl.ds(o, 16)] = d2
            id1[pl.ds(o, 16)] = d2 + 1
        pltpu.sync_copy(ones_v, acc_sp.at[id0], add=True)
        pltpu.sync_copy(ones_v, acc_sp.at[id1], add=True)
        return carry

    lax.fori_loop(0, QB, body, 0)
    plsc.subcore_barrier()
    pltpu.sync_copy(acc_sp.at[pl.ds(r0, SLICE)], out_hbm.at[cid, pl.ds(r0, SLICE)])


def _tc_idx_body(ei_ref, s_ref, d_ref):
    r = lax.broadcasted_iota(jnp.int32, s_ref.shape, 0)
    l = lax.broadcasted_iota(jnp.int32, s_ref.shape, 1)
    dm = N + ((r * 128 + l) % 2048)
    e0 = ei_ref[0]
    e1 = ei_ref[1]
    s_ref[...] = jnp.where(e0 < N, e0, dm)
    d_ref[...] = jnp.where(e1 < N, e1, dm)


def _parity_masks(shape):
    lane = lax.broadcasted_iota(jnp.int32, shape, len(shape) - 1)
    mo = (lane % 2).astype(jnp.float32)
    return 1.0 - mo, mo


def _tc_a_body(degp_ref, x_ref, dinv_ref, xs1_ref):
    deg = degp_ref[0] + degp_ref[1] + 1.0
    dinv = lax.rsqrt(deg)
    dinv_ref[...] = dinv
    xs1_ref[...] = x_ref[...] * dinv


def _tc_b_body(w1_ref, b1_ref, w2_ref, aggp_ref, xs1_ref, dinv_ref, xs2_ref):
    dinv = dinv_ref[...]
    t = dinv * (aggp_ref[0] + aggp_ref[1] + xs1_ref[...])
    me, mo = _parity_masks(t.shape)
    tr = pltpu.roll(t, 127, axis=1)
    tl = pltpu.roll(t, 1, axis=1)
    t0 = me * t + mo * tl
    t1 = me * tr + mo * t
    y0 = jnp.zeros_like(t)
    y1 = jnp.zeros_like(t)
    for j in range(64):
        h = jnp.maximum(t0 * w1_ref[0, j] + t1 * w1_ref[1, j] + b1_ref[j], 0.0)
        y0 = y0 + h * w2_ref[j, 0]
        y1 = y1 + h * w2_ref[j, 1]
    xs2_ref[...] = dinv * (me * y0 + mo * y1)


def _tc_c_body(b2_ref, aggp_ref, xs2_ref, dinv_ref, out_ref):
    me, mo = _parity_masks(xs2_ref.shape)
    out_ref[...] = (dinv_ref[...] * (aggp_ref[0] + aggp_ref[1] + xs2_ref[...])
                    + me * b2_ref[0] + mo * b2_ref[1])


def _il_spec(lead):
    if lead:
        return pl.BlockSpec((*lead, IL_BLK, 128),
                            lambda i: (*([0] * len(lead)), i, 0))
    return pl.BlockSpec((IL_BLK, 128), lambda i: (i, 0))


_SMEM = pl.BlockSpec(memory_space=pltpu.SMEM)
_IL1 = jax.ShapeDtypeStruct((IL_ROWS, 128), jnp.float32)

IDX_BLK = 448
IDX_GRID = EROWS_P // IDX_BLK

_tc_idx = pl.pallas_call(
    _tc_idx_body,
    grid=(IDX_GRID,),
    in_specs=[pl.BlockSpec((2, IDX_BLK, 128), lambda i: (0, i, 0))],
    out_specs=[pl.BlockSpec((IDX_BLK, 128), lambda i: (i, 0))] * 2,
    out_shape=[jax.ShapeDtypeStruct((EROWS_P, 128), jnp.int32)] * 2,
)

_tc_a = pl.pallas_call(
    _tc_a_body,
    grid=(IL_GRID,),
    in_specs=[_il_spec((2,)), _il_spec(())],
    out_specs=[_il_spec(()), _il_spec(())],
    out_shape=[_IL1, _IL1],
)

_tc_b = pl.pallas_call(
    _tc_b_body,
    grid=(IL_GRID,),
    in_specs=[_SMEM, _SMEM, _SMEM, _il_spec((2,)), _il_spec(()), _il_spec(())],
    out_specs=[_il_spec(())],
    out_shape=[_IL1],
)

_tc_c = pl.pallas_call(
    _tc_c_body,
    grid=(IL_GRID,),
    in_specs=[_SMEM, _il_spec((2,)), _il_spec(()), _il_spec(())],
    out_specs=[_il_spec(())],
    out_shape=[_IL1],
)


def kernel(x, edge_index, W1, b1, W2, b2):
    ei3 = edge_index.astype(jnp.int32).reshape(2, EROWS, 128)
    eip = jnp.pad(ei3, ((0, 0), (0, EROWS_P - EROWS), (0, 0)),
                  constant_values=N)
    srcT, dstT = _tc_idx(eip)
    zeros_flat = jnp.zeros((FLAT,), jnp.float32)
    ones_flat = jnp.ones((FLAT,), jnp.float32)

    degp = _sc_deg(ones_flat, dstT, zeros_flat)
    degp_il = degp.reshape(2, IL_ROWS, 128)
    x_il = jnp.pad(x, ((0, NPAD - N), (0, 0))).reshape(IL_ROWS, 128)
    dinv_il, xs1_il = _tc_a(degp_il, x_il)

    agg1p = _sc_agg(xs1_il.reshape(FLAT), srcT, dstT, zeros_flat)
    (xs2_il,) = _tc_b(W1, b1, W2, agg1p.reshape(2, IL_ROWS, 128),
                      xs1_il, dinv_il)

    agg2p = _sc_agg(xs2_il.reshape(FLAT), srcT, dstT, zeros_flat)
    (out_il,) = _tc_c(b2, agg2p.reshape(2, IL_ROWS, 128), xs2_il, dinv_il)

    return out_il.reshape(NPAD, 2)[:N]


# speedup vs baseline: 3.9949x; 1.0131x over previous
import functools

import jax
import jax.numpy as jnp
from jax import lax
from jax.experimental import pallas as pl
from jax.experimental.pallas import tpu as pltpu
from jax.experimental.pallas import tpu_sc as plsc

N = 100000
NPAD = 102400
E = 1600000
FLAT = 2 * NPAD
SLICE = FLAT // 16
EROWS = E // 128
EROWS_P = 12544
WROWS = EROWS_P // 32
QB = 49
EB = 1024
IL_ROWS = FLAT // 128
IL_BLK = 80
IL_GRID = IL_ROWS // IL_BLK

_mesh = plsc.VectorSubcoreMesh(
    core_axis_name="c", subcore_axis_name="s", num_cores=2, num_subcores=16
)


@functools.partial(
    pl.kernel,
    out_type=jax.ShapeDtypeStruct((2, FLAT), jnp.float32),
    mesh=_mesh,
    scratch_types=[
        pltpu.VMEM_SHARED((FLAT,), jnp.float32),
        pltpu.VMEM_SHARED((FLAT,), jnp.float32),
        pltpu.VMEM((8, 128), jnp.int32),
        pltpu.VMEM((8, 128), jnp.int32),
        pltpu.VMEM((EB,), jnp.int32),
        pltpu.VMEM((EB,), jnp.int32),
        pltpu.VMEM((EB,), jnp.int32),
        pltpu.VMEM((EB,), jnp.int32),
        pltpu.VMEM((EB,), jnp.float32),
        pltpu.VMEM((EB,), jnp.float32),
        pltpu.SemaphoreType.DMA,
        pltpu.SemaphoreType.DMA,
    ],
)
def _sc_agg(xs_hbm, src_hbm, dst_hbm, zeros_hbm, out_hbm,
            xs_sp, acc_sp, sbuf, dbuf, ie0, ie1, id0, id1, ve, vo,
            sem_e, sem_o):
    cid = lax.axis_index("c")
    sid = lax.axis_index("s")
    r0 = sid * SLICE
    w = cid * 16 + sid
    pltpu.sync_copy(xs_hbm.at[pl.ds(r0, SLICE)], xs_sp.at[pl.ds(r0, SLICE)])
    pltpu.sync_copy(zeros_hbm.at[pl.ds(r0, SLICE)], acc_sp.at[pl.ds(r0, SLICE)])
    plsc.subcore_barrier()

    row0 = w * WROWS

    def body(q, carry):
        r = row0 + q * 8
        pltpu.sync_copy(src_hbm.at[pl.ds(r, 8)], sbuf)
        pltpu.sync_copy(dst_hbm.at[pl.ds(r, 8)], dbuf)
        for j in range(64):
            ri, c = j // 8, (j % 8) * 16
            o = j * 16
            s2 = sbuf[ri, pl.ds(c, 16)]
            s2 = s2 + s2
            ie0[pl.ds(o, 16)] = s2
            ie1[pl.ds(o, 16)] = s2 + 1
            d2 = dbuf[ri, pl.ds(c, 16)]
            d2 = d2 + d2
            id0[pl.ds(o, 16)] = d2
            id1[pl.ds(o, 16)] = d2 + 1
        cpe = pltpu.async_copy(xs_sp.at[ie0], ve, sem_e)
        cpo = pltpu.async_copy(xs_sp.at[ie1], vo, sem_o)
        cpe.wait()
        cpo.wait()
        cse = pltpu.async_copy(ve, acc_sp.at[id0], sem_e, add=True)
        cso = pltpu.async_copy(vo, acc_sp.at[id1], sem_o, add=True)
        cse.wait()
        cso.wait()
        return carry

    lax.fori_loop(0, QB, body, 0)
    plsc.subcore_barrier()
    pltpu.sync_copy(acc_sp.at[pl.ds(r0, SLICE)], out_hbm.at[cid, pl.ds(r0, SLICE)])


@functools.partial(
    pl.kernel,
    out_type=jax.ShapeDtypeStruct((2, FLAT), jnp.float32),
    mesh=_mesh,
    scratch_types=[
        pltpu.VMEM_SHARED((FLAT,), jnp.float32),
        pltpu.VMEM((8, 128), jnp.int32),
        pltpu.VMEM((EB,), jnp.int32),
        pltpu.VMEM((EB,), jnp.int32),
        pltpu.VMEM((EB,), jnp.float32),
        pltpu.SemaphoreType.DMA,
        pltpu.SemaphoreType.DMA,
    ],
)
def _sc_deg(ones_hbm, dst_hbm, zeros_hbm, out_hbm,
            acc_sp, dbuf, id0, id1, ones_v, sem_e, sem_o):
    cid = lax.axis_index("c")
    sid = lax.axis_index("s")
    r0 = sid * SLICE
    w = cid * 16 + sid
    pltpu.sync_copy(zeros_hbm.at[pl.ds(r0, SLICE)], acc_sp.at[pl.ds(r0, SLICE)])
    pltpu.sync_copy(ones_hbm.at[pl.ds(0, EB)], ones_v)
    plsc.subcore_barrier()

    row0 = w * WROWS

    def body(q, carry):
        r = row0 + q * 8
        pltpu.sync_copy(dst_hbm.at[pl.ds(r, 8)], dbuf)
        for j in range(64):
            ri, c = j // 8, (j % 8) * 16
            o = j * 16
            d2 = dbuf[ri, pl.ds(c, 16)]
            d2 = d2 + d2
            id0[pl.ds(o, 16)] = d2
            id1[pl.ds(o, 16)] = d2 + 1
        cse = pltpu.async_copy(ones_v, acc_sp.at[id0], sem_e, add=True)
        cso = pltpu.async_copy(ones_v, acc_sp.at[id1], sem_o, add=True)
        cse.wait()
        cso.wait()
        return carry

    lax.fori_loop(0, QB, body, 0)
    plsc.subcore_barrier()
    pltpu.sync_copy(acc_sp.at[pl.ds(r0, SLICE)], out_hbm.at[cid, pl.ds(r0, SLICE)])


def _tc_idx_body(ei_ref, s_ref, d_ref):
    r = lax.broadcasted_iota(jnp.int32, s_ref.shape, 0)
    l = lax.broadcasted_iota(jnp.int32, s_ref.shape, 1)
    dm = N + ((r * 128 + l) % 2048)
    e0 = ei_ref[0]
    e1 = ei_ref[1]
    s_ref[...] = jnp.where(e0 < N, e0, dm)
    d_ref[...] = jnp.where(e1 < N, e1, dm)


def _parity_masks(shape):
    lane = lax.broadcasted_iota(jnp.int32, shape, len(shape) - 1)
    mo = (lane % 2).astype(jnp.float32)
    return 1.0 - mo, mo


def _tc_a_body(degp_ref, x_ref, dinv_ref, xs1_ref):
    deg = degp_ref[0] + degp_ref[1] + 1.0
    dinv = lax.rsqrt(deg)
    dinv_ref[...] = dinv
    xs1_ref[...] = x_ref[...] * dinv


def _tc_b_body(w1_ref, b1_ref, w2_ref, aggp_ref, xs1_ref, dinv_ref, xs2_ref):
    dinv = dinv_ref[...]
    t = dinv * (aggp_ref[0] + aggp_ref[1] + xs1_ref[...])
    me, mo = _parity_masks(t.shape)
    tr = pltpu.roll(t, 127, axis=1)
    tl = pltpu.roll(t, 1, axis=1)
    t0 = me * t + mo * tl
    t1 = me * tr + mo * t
    y0 = jnp.zeros_like(t)
    y1 = jnp.zeros_like(t)
    for j in range(64):
        h = jnp.maximum(t0 * w1_ref[0, j] + t1 * w1_ref[1, j] + b1_ref[j], 0.0)
        y0 = y0 + h * w2_ref[j, 0]
        y1 = y1 + h * w2_ref[j, 1]
    xs2_ref[...] = dinv * (me * y0 + mo * y1)


def _tc_c_body(b2_ref, aggp_ref, xs2_ref, dinv_ref, out_ref):
    me, mo = _parity_masks(xs2_ref.shape)
    out_ref[...] = (dinv_ref[...] * (aggp_ref[0] + aggp_ref[1] + xs2_ref[...])
                    + me * b2_ref[0] + mo * b2_ref[1])


def _il_spec(lead):
    if lead:
        return pl.BlockSpec((*lead, IL_BLK, 128),
                            lambda i: (*([0] * len(lead)), i, 0))
    return pl.BlockSpec((IL_BLK, 128), lambda i: (i, 0))


_SMEM = pl.BlockSpec(memory_space=pltpu.SMEM)
_IL1 = jax.ShapeDtypeStruct((IL_ROWS, 128), jnp.float32)

IDX_BLK = 448
IDX_GRID = EROWS_P // IDX_BLK

_tc_idx = pl.pallas_call(
    _tc_idx_body,
    grid=(IDX_GRID,),
    in_specs=[pl.BlockSpec((2, IDX_BLK, 128), lambda i: (0, i, 0))],
    out_specs=[pl.BlockSpec((IDX_BLK, 128), lambda i: (i, 0))] * 2,
    out_shape=[jax.ShapeDtypeStruct((EROWS_P, 128), jnp.int32)] * 2,
)

_tc_a = pl.pallas_call(
    _tc_a_body,
    grid=(IL_GRID,),
    in_specs=[_il_spec((2,)), _il_spec(())],
    out_specs=[_il_spec(()), _il_spec(())],
    out_shape=[_IL1, _IL1],
)

_tc_b = pl.pallas_call(
    _tc_b_body,
    grid=(IL_GRID,),
    in_specs=[_SMEM, _SMEM, _SMEM, _il_spec((2,)), _il_spec(()), _il_spec(())],
    out_specs=[_il_spec(())],
    out_shape=[_IL1],
)

_tc_c = pl.pallas_call(
    _tc_c_body,
    grid=(IL_GRID,),
    in_specs=[_SMEM, _il_spec((2,)), _il_spec(()), _il_spec(())],
    out_specs=[_il_spec(())],
    out_shape=[_IL1],
)


def kernel(x, edge_index, W1, b1, W2, b2):
    ei3 = edge_index.astype(jnp.int32).reshape(2, EROWS, 128)
    eip = jnp.pad(ei3, ((0, 0), (0, EROWS_P - EROWS), (0, 0)),
                  constant_values=N)
    srcT, dstT = _tc_idx(eip)
    zeros_flat = jnp.zeros((FLAT,), jnp.float32)
    ones_flat = jnp.ones((FLAT,), jnp.float32)

    degp = _sc_deg(ones_flat, dstT, zeros_flat)
    degp_il = degp.reshape(2, IL_ROWS, 128)
    x_il = jnp.pad(x, ((0, NPAD - N), (0, 0))).reshape(IL_ROWS, 128)
    dinv_il, xs1_il = _tc_a(degp_il, x_il)

    agg1p = _sc_agg(xs1_il.reshape(FLAT), srcT, dstT, zeros_flat)
    (xs2_il,) = _tc_b(W1, b1, W2, agg1p.reshape(2, IL_ROWS, 128),
                      xs1_il, dinv_il)

    agg2p = _sc_agg(xs2_il.reshape(FLAT), srcT, dstT, zeros_flat)
    (out_il,) = _tc_c(b2, agg2p.reshape(2, IL_ROWS, 128), xs2_il, dinv_il)

    return out_il.reshape(NPAD, 2)[:N]
